# baseline (speedup 1.0000x reference)
import functools

import jax
import jax.numpy as jnp
from jax import lax
from jax.experimental import pallas as pl
from jax.experimental.pallas import tpu as pltpu
from jax.experimental.pallas import tpu_sc as plsc

BATCH = 16384
EMBED_DIM = 64

_info = plsc.get_sparse_core_info()
_NC, _NS = _info.num_cores, _info.num_subcores
_NW = _NC * _NS
_B_PER_W = BATCH // _NW


def _gather_body(ut_hbm, it_hbm, uidx_hbm, iidx_hbm, eu_hbm, ei_hbm,
                 uidx_v, iidx_v, urows_v, irows_v, sem_u, sem_i):
    wid = lax.axis_index("s") * _NC + lax.axis_index("c")
    base = wid * _B_PER_W
    pltpu.sync_copy(uidx_hbm.at[pl.ds(base, _B_PER_W)], uidx_v)
    pltpu.sync_copy(iidx_hbm.at[pl.ds(base, _B_PER_W)], iidx_v)
    cu = pltpu.async_copy(ut_hbm.at[uidx_v], urows_v, sem_u)
    ci = pltpu.async_copy(it_hbm.at[iidx_v], irows_v, sem_i)
    cu.wait()
    ci.wait()
    pltpu.sync_copy(urows_v, eu_hbm.at[pl.ds(base, _B_PER_W)])
    pltpu.sync_copy(irows_v, ei_hbm.at[pl.ds(base, _B_PER_W)])


_sc_gather = functools.partial(
    pl.kernel,
    mesh=plsc.VectorSubcoreMesh(core_axis_name="c", subcore_axis_name="s"),
    out_type=[
        jax.ShapeDtypeStruct((BATCH, EMBED_DIM), jnp.float32),
        jax.ShapeDtypeStruct((BATCH, EMBED_DIM), jnp.float32),
    ],
    scratch_types=[
        pltpu.VMEM((_B_PER_W,), jnp.int32),
        pltpu.VMEM((_B_PER_W,), jnp.int32),
        pltpu.VMEM((_B_PER_W, EMBED_DIM), jnp.float32),
        pltpu.VMEM((_B_PER_W, EMBED_DIM), jnp.float32),
        pltpu.SemaphoreType.DMA,
        pltpu.SemaphoreType.DMA,
    ],
    compiler_params=pltpu.CompilerParams(use_tc_tiling_on_sc=False),
)(_gather_body)


def _mlp_body(eu_ref, ei_ref, w1a_ref, w1b_ref, b1_ref, w2_ref, b2_ref,
              w3_ref, b3_ref, wp_ref, bp_ref, out_ref):
    h = eu_ref[...] @ w1a_ref[...] + ei_ref[...] @ w1b_ref[...] + b1_ref[...]
    h = jnp.maximum(h, 0.0)
    h = jnp.maximum(h @ w2_ref[...] + b2_ref[...], 0.0)
    h = jnp.maximum(h @ w3_ref[...] + b3_ref[...], 0.0)
    logit = jnp.sum(h * wp_ref[...], axis=1) + bp_ref[0]
    out_ref[...] = jax.nn.sigmoid(logit)


def _mlp(eu, ei, w1a, w1b, b1, w2, b2, w3, b3, wp, bp, blk=2048):
    n_blocks = BATCH // blk
    def full(shape):
        zeros = (0,) * len(shape)
        return pl.BlockSpec(shape, lambda i: zeros)
    return pl.pallas_call(
        _mlp_body,
        grid=(n_blocks,),
        in_specs=[
            pl.BlockSpec((blk, EMBED_DIM), lambda i: (i, 0)),
            pl.BlockSpec((blk, EMBED_DIM), lambda i: (i, 0)),
            full(w1a.shape),
            full(w1b.shape),
            full(b1.shape),
            full(w2.shape),
            full(b2.shape),
            full(w3.shape),
            full(b3.shape),
            full(wp.shape),
            full(bp.shape),
        ],
        out_specs=pl.BlockSpec((blk,), lambda i: (i,)),
        out_shape=jax.ShapeDtypeStruct((BATCH,), jnp.float32),
    )(eu, ei, w1a, w1b, b1, w2, b2, w3, b3, wp, bp)


def kernel(user, item, embed_user, embed_item, W1, b1, W2, b2, W3, b3, Wp, bp):
    u = user.astype(jnp.int32)
    it = item.astype(jnp.int32)
    eu, ei = _sc_gather(embed_user, embed_item, u, it)
    w1a = W1[:, :EMBED_DIM].T
    w1b = W1[:, EMBED_DIM:].T
    return _mlp(
        eu, ei, w1a, w1b,
        b1.reshape(1, -1), W2.T, b2.reshape(1, -1), W3.T, b3.reshape(1, -1),
        Wp, bp,
    )


# speedup vs baseline: 1.5718x; 1.5718x over previous
import functools

import jax
import jax.numpy as jnp
from jax import lax
from jax.experimental import pallas as pl
from jax.experimental.pallas import tpu as pltpu
from jax.experimental.pallas import tpu_sc as plsc

BATCH = 16384
EMBED_DIM = 64

_info = plsc.get_sparse_core_info()
_NC, _NS = _info.num_cores, _info.num_subcores
_NW = _NC * _NS
_B_PER_W = BATCH // _NW


_CHUNK = 128
_N_CHUNKS = _B_PER_W // _CHUNK


def _gather_body(ut_hbm, it_hbm, uidx_hbm, iidx_hbm, eu_hbm, ei_hbm,
                 uidx_v, iidx_v,
                 ubuf0, ubuf1, ibuf0, ibuf1,
                 sem_u0, sem_u1, sem_i0, sem_i1):
    wid = lax.axis_index("s") * _NC + lax.axis_index("c")
    base = wid * _B_PER_W
    pltpu.sync_copy(uidx_hbm.at[pl.ds(base, _B_PER_W)], uidx_v)
    pltpu.sync_copy(iidx_hbm.at[pl.ds(base, _B_PER_W)], iidx_v)

    ubufs = (ubuf0, ubuf1)
    ibufs = (ibuf0, ibuf1)
    usems = (sem_u0, sem_u1)
    isems = (sem_i0, sem_i1)

    def fire(c, ub, ib, us, se):
        off = c * _CHUNK

        def body(g, _):
            uvec = uidx_v[pl.ds(off + g * 16, 16)]
            ivec = iidx_v[pl.ds(off + g * 16, 16)]
            for k in range(16):
                pltpu.async_copy(ut_hbm.at[uvec[k]], ub.at[g * 16 + k], us)
                pltpu.async_copy(it_hbm.at[ivec[k]], ib.at[g * 16 + k], se)
            return 0

        lax.fori_loop(0, _CHUNK // 16, body, 0)

    def drain(ub, ib, us, se):
        def body(i, _):
            pltpu.make_async_copy(ut_hbm.at[0], ub.at[0], us).wait()
            pltpu.make_async_copy(it_hbm.at[0], ib.at[0], se).wait()
            return 0

        lax.fori_loop(0, _CHUNK, body, 0, unroll=8)

    fire(0, ubufs[0], ibufs[0], usems[0], isems[0])
    for c in range(1, _N_CHUNKS):
        p, q = c % 2, (c - 1) % 2
        fire(c, ubufs[p], ibufs[p], usems[p], isems[p])
        drain(ubufs[q], ibufs[q], usems[q], isems[q])
        out = base + (c - 1) * _CHUNK
        pltpu.sync_copy(ubufs[q], eu_hbm.at[pl.ds(out, _CHUNK)])
        pltpu.sync_copy(ibufs[q], ei_hbm.at[pl.ds(out, _CHUNK)])
    q = (_N_CHUNKS - 1) % 2
    drain(ubufs[q], ibufs[q], usems[q], isems[q])
    out = base + (_N_CHUNKS - 1) * _CHUNK
    pltpu.sync_copy(ubufs[q], eu_hbm.at[pl.ds(out, _CHUNK)])
    pltpu.sync_copy(ibufs[q], ei_hbm.at[pl.ds(out, _CHUNK)])


_sc_gather = functools.partial(
    pl.kernel,
    mesh=plsc.VectorSubcoreMesh(core_axis_name="c", subcore_axis_name="s"),
    out_type=[
        jax.ShapeDtypeStruct((BATCH, EMBED_DIM), jnp.float32),
        jax.ShapeDtypeStruct((BATCH, EMBED_DIM), jnp.float32),
    ],
    scratch_types=[
        pltpu.VMEM((_B_PER_W,), jnp.int32),
        pltpu.VMEM((_B_PER_W,), jnp.int32),
        pltpu.VMEM((_CHUNK, EMBED_DIM), jnp.float32),
        pltpu.VMEM((_CHUNK, EMBED_DIM), jnp.float32),
        pltpu.VMEM((_CHUNK, EMBED_DIM), jnp.float32),
        pltpu.VMEM((_CHUNK, EMBED_DIM), jnp.float32),
        pltpu.SemaphoreType.DMA,
        pltpu.SemaphoreType.DMA,
        pltpu.SemaphoreType.DMA,
        pltpu.SemaphoreType.DMA,
    ],
)(_gather_body)


def _mlp_body(eu_ref, ei_ref, w1a_ref, w1b_ref, b1_ref, w2_ref, b2_ref,
              w3_ref, b3_ref, wp_ref, bp_ref, out_ref):
    h = eu_ref[...] @ w1a_ref[...] + ei_ref[...] @ w1b_ref[...] + b1_ref[...]
    h = jnp.maximum(h, 0.0)
    h = jnp.maximum(h @ w2_ref[...] + b2_ref[...], 0.0)
    h = jnp.maximum(h @ w3_ref[...] + b3_ref[...], 0.0)
    logit = jnp.sum(h * wp_ref[...], axis=1) + bp_ref[0]
    out_ref[...] = jax.nn.sigmoid(logit)


def _mlp(eu, ei, w1a, w1b, b1, w2, b2, w3, b3, wp, bp, blk=2048):
    n_blocks = BATCH // blk
    def full(shape):
        zeros = (0,) * len(shape)
        return pl.BlockSpec(shape, lambda i: zeros)
    return pl.pallas_call(
        _mlp_body,
        grid=(n_blocks,),
        in_specs=[
            pl.BlockSpec((blk, EMBED_DIM), lambda i: (i, 0)),
            pl.BlockSpec((blk, EMBED_DIM), lambda i: (i, 0)),
            full(w1a.shape),
            full(w1b.shape),
            full(b1.shape),
            full(w2.shape),
            full(b2.shape),
            full(w3.shape),
            full(b3.shape),
            full(wp.shape),
            full(bp.shape),
        ],
        out_specs=pl.BlockSpec((blk,), lambda i: (i,)),
        out_shape=jax.ShapeDtypeStruct((BATCH,), jnp.float32),
    )(eu, ei, w1a, w1b, b1, w2, b2, w3, b3, wp, bp)


def kernel(user, item, embed_user, embed_item, W1, b1, W2, b2, W3, b3, Wp, bp):
    u = user.astype(jnp.int32)
    it = item.astype(jnp.int32)
    eu, ei = _sc_gather(embed_user, embed_item, u, it)
    w1a = W1[:, :EMBED_DIM].T
    w1b = W1[:, EMBED_DIM:].T
    return _mlp(
        eu, ei, w1a, w1b,
        b1.reshape(1, -1), W2.T, b2.reshape(1, -1), W3.T, b3.reshape(1, -1),
        Wp, bp,
    )


# speedup vs baseline: 2.4310x; 1.5466x over previous
import functools

import jax
import jax.numpy as jnp
from jax import lax
from jax.experimental import pallas as pl
from jax.experimental.pallas import tpu as pltpu
from jax.experimental.pallas import tpu_sc as plsc

BATCH = 16384
EMBED_DIM = 64
PANEL = 128

_info = plsc.get_sparse_core_info()
_NC, _NS = _info.num_cores, _info.num_subcores
_NW = _NC * _NS
_B_PER_W = BATCH // _NW

_N_SLOTS = 4
_STAGE_W = 128
_N_STAGES = _B_PER_W // _STAGE_W


def _gather_body(utT_hbm, itT_hbm, uidx_hbm, iidx_hbm, euT_hbm, eiT_hbm,
                 uidx_v, iidx_v, pb0, pb1, pb2, pb3, stage,
                 sem0, sem1, sem2, sem3):
    wid = lax.axis_index("s") * _NC + lax.axis_index("c")
    base = pl.multiple_of(wid * _B_PER_W, _B_PER_W)
    pltpu.sync_copy(uidx_hbm.at[pl.ds(base, _B_PER_W)], uidx_v)
    pltpu.sync_copy(iidx_hbm.at[pl.ds(base, _B_PER_W)], iidx_v)

    pbs = (pb0, pb1, pb2, pb3)
    sems = (sem0, sem1, sem2, sem3)
    iota = lax.broadcasted_iota(jnp.int32, (16,), 0)

    def fire(tab, r, pb, sem):
        poff = pl.multiple_of(r - (r & (PANEL - 1)), PANEL)
        pltpu.async_copy(tab.at[:, pl.ds(poff, PANEL)], pb, sem)

    def wait(tab, pb, sem):
        pltpu.make_async_copy(tab.at[:, pl.ds(0, PANEL)], pb, sem).wait()

    def phase(tab, idxv, outT):
        v0 = idxv[pl.ds(0, 16)]
        for k in range(_N_SLOTS):
            fire(tab, v0[k], pbs[k], sems[k])

        cur0 = v0
        for chunk in range(_N_STAGES):
            def group_body(g, cur, chunk=chunk):
                t0 = chunk * _STAGE_W + g * 16
                nxt = idxv[pl.ds(jnp.minimum(t0 + 16, _B_PER_W - 16), 16)]
                for k in range(16):
                    s = k % _N_SLOTS
                    wait(tab, pbs[s], sems[s])
                    r = cur[k]
                    c = jnp.broadcast_to(r & (PANEL - 1), (16,))
                    cc = jnp.broadcast_to(g * 16 + k, (16,))
                    for m in range(4):
                        rows = iota + 16 * m
                        v = plsc.load_gather(pbs[s], [rows, c])
                        plsc.store_scatter(stage, [rows, cc], v)
                    tt = t0 + k + _N_SLOTS
                    rn = cur[k + _N_SLOTS] if k < 16 - _N_SLOTS else nxt[k - (16 - _N_SLOTS)]

                    @pl.when(tt < _B_PER_W)
                    def _():
                        fire(tab, rn, pbs[s], sems[s])
                return nxt

            cur0 = lax.fori_loop(0, _STAGE_W // 16, group_body, cur0)
            out = pl.multiple_of(base + chunk * _STAGE_W, _STAGE_W)
            pltpu.sync_copy(stage, outT.at[:, pl.ds(out, _STAGE_W)])

    phase(utT_hbm, uidx_v, euT_hbm)
    phase(itT_hbm, iidx_v, eiT_hbm)


_sc_gather = functools.partial(
    pl.kernel,
    mesh=plsc.VectorSubcoreMesh(core_axis_name="c", subcore_axis_name="s"),
    out_type=[
        jax.ShapeDtypeStruct((EMBED_DIM, BATCH), jnp.float32),
        jax.ShapeDtypeStruct((EMBED_DIM, BATCH), jnp.float32),
    ],
    scratch_types=[
        pltpu.VMEM((_B_PER_W,), jnp.int32),
        pltpu.VMEM((_B_PER_W,), jnp.int32),
        pltpu.VMEM((EMBED_DIM, PANEL), jnp.float32),
        pltpu.VMEM((EMBED_DIM, PANEL), jnp.float32),
        pltpu.VMEM((EMBED_DIM, PANEL), jnp.float32),
        pltpu.VMEM((EMBED_DIM, PANEL), jnp.float32),
        pltpu.VMEM((EMBED_DIM, _STAGE_W), jnp.float32),
        pltpu.SemaphoreType.DMA,
        pltpu.SemaphoreType.DMA,
        pltpu.SemaphoreType.DMA,
        pltpu.SemaphoreType.DMA,
    ],
    compiler_params=pltpu.CompilerParams(needs_layout_passes=False),
)(_gather_body)


def _mlp_body(euT_ref, eiT_ref, w1a_ref, w1b_ref, b1_ref, w2_ref, b2_ref,
              w3_ref, b3_ref, wp_ref, bp_ref, out_ref):
    h = w1a_ref[...] @ euT_ref[...] + w1b_ref[...] @ eiT_ref[...] + b1_ref[...]
    h = jnp.maximum(h, 0.0)
    h = jnp.maximum(w2_ref[...] @ h + b2_ref[...], 0.0)
    h = jnp.maximum(w3_ref[...] @ h + b3_ref[...], 0.0)
    logit = wp_ref[...] @ h + bp_ref[...]
    out_ref[...] = jax.nn.sigmoid(logit)


def _mlp(euT, eiT, w1a, w1b, b1, w2, b2, w3, b3, wp, bp, blk=4096):
    n_blocks = BATCH // blk

    def full(shape):
        zeros = (0,) * len(shape)
        return pl.BlockSpec(shape, lambda i: zeros)

    return pl.pallas_call(
        _mlp_body,
        grid=(n_blocks,),
        in_specs=[
            pl.BlockSpec((EMBED_DIM, blk), lambda i: (0, i)),
            pl.BlockSpec((EMBED_DIM, blk), lambda i: (0, i)),
            full(w1a.shape),
            full(w1b.shape),
            full(b1.shape),
            full(w2.shape),
            full(b2.shape),
            full(w3.shape),
            full(b3.shape),
            full(wp.shape),
            full(bp.shape),
        ],
        out_specs=pl.BlockSpec((1, blk), lambda i: (0, i)),
        out_shape=jax.ShapeDtypeStruct((1, BATCH), jnp.float32),
    )(euT, eiT, w1a, w1b, b1, w2, b2, w3, b3, wp, bp)


def kernel(user, item, embed_user, embed_item, W1, b1, W2, b2, W3, b3, Wp, bp):
    u = user.astype(jnp.int32)
    it = item.astype(jnp.int32)
    euT, eiT = _sc_gather(embed_user.T, embed_item.T, u, it)
    out = _mlp(
        euT, eiT,
        W1[:, :EMBED_DIM], W1[:, EMBED_DIM:], b1.reshape(-1, 1),
        W2, b2.reshape(-1, 1), W3, b3.reshape(-1, 1), Wp, bp.reshape(1, 1),
    )
    return out.reshape(-1)


# speedup vs baseline: 2.6654x; 1.0965x over previous
import functools

import jax
import jax.numpy as jnp
from jax import lax
from jax.experimental import pallas as pl
from jax.experimental.pallas import tpu as pltpu
from jax.experimental.pallas import tpu_sc as plsc

BATCH = 16384
EMBED_DIM = 64
PANEL = 128

_info = plsc.get_sparse_core_info()
_NC, _NS = _info.num_cores, _info.num_subcores
_NW = _NC * _NS
_B_PER_W = BATCH // _NW

_N_SLOTS = 8
_PANEL_H = 32
_STAGE_W = 128
_N_STAGES = _B_PER_W // _STAGE_W


def _gather_body(utT_hbm, itT_hbm, uidx_hbm, iidx_hbm, euT_hbm, eiT_hbm,
                 uidx_v, iidx_v, pb0, pb1, pb2, pb3, pb4, pb5, pb6, pb7,
                 stage, sem0, sem1, sem2, sem3, sem4, sem5, sem6, sem7):
    wid = lax.axis_index("s") * _NC + lax.axis_index("c")
    base = pl.multiple_of(wid * _B_PER_W, _B_PER_W)
    pltpu.sync_copy(uidx_hbm.at[pl.ds(base, _B_PER_W)], uidx_v)
    pltpu.sync_copy(iidx_hbm.at[pl.ds(base, _B_PER_W)], iidx_v)

    pbs = (pb0, pb1, pb2, pb3, pb4, pb5, pb6, pb7)
    sems = (sem0, sem1, sem2, sem3, sem4, sem5, sem6, sem7)
    iota = lax.broadcasted_iota(jnp.int32, (16,), 0)

    def phase(tab, idxv, outT, h0):
        def fire(r, pb, sem):
            poff = pl.multiple_of(r - (r & (PANEL - 1)), PANEL)
            pltpu.async_copy(tab.at[pl.ds(h0, _PANEL_H), pl.ds(poff, PANEL)],
                             pb, sem)

        def wait(pb, sem):
            pltpu.make_async_copy(tab.at[pl.ds(h0, _PANEL_H), pl.ds(0, PANEL)],
                                  pb, sem).wait()

        v0 = idxv[pl.ds(0, 16)]
        for k in range(_N_SLOTS):
            fire(v0[k], pbs[k], sems[k])

        groups_per_stage = _STAGE_W // 16

        def group_body(g, cur):
            t0 = g * 16
            nxt = idxv[pl.ds(jnp.minimum(t0 + 16, _B_PER_W - 16), 16)]
            for k in range(16):
                s = k % _N_SLOTS
                wait(pbs[s], sems[s])
                r = cur[k]
                c = jnp.broadcast_to(r & (PANEL - 1), (16,))
                cc = jnp.broadcast_to((g % groups_per_stage) * 16 + k, (16,))
                for m in range(_PANEL_H // 16):
                    rows = iota + 16 * m
                    v = plsc.load_gather(pbs[s], [rows, c])
                    plsc.store_scatter(stage, [rows, cc], v)
                tt = t0 + k + _N_SLOTS
                rn = cur[k + _N_SLOTS] if k < 16 - _N_SLOTS else nxt[k - (16 - _N_SLOTS)]

                @pl.when(tt < _B_PER_W)
                def _():
                    fire(rn, pbs[s], sems[s])

            @pl.when(g % groups_per_stage == groups_per_stage - 1)
            def _():
                out = pl.multiple_of(
                    base + (g // groups_per_stage) * _STAGE_W, _STAGE_W)
                pltpu.sync_copy(
                    stage, outT.at[pl.ds(h0, _PANEL_H), pl.ds(out, _STAGE_W)])
            return nxt

        lax.fori_loop(0, _B_PER_W // 16, group_body, v0)

    for h0 in range(0, EMBED_DIM, _PANEL_H):
        phase(utT_hbm, uidx_v, euT_hbm, h0)
    for h0 in range(0, EMBED_DIM, _PANEL_H):
        phase(itT_hbm, iidx_v, eiT_hbm, h0)


_sc_gather = functools.partial(
    pl.kernel,
    mesh=plsc.VectorSubcoreMesh(core_axis_name="c", subcore_axis_name="s"),
    out_type=[
        jax.ShapeDtypeStruct((EMBED_DIM, BATCH), jnp.float32),
        jax.ShapeDtypeStruct((EMBED_DIM, BATCH), jnp.float32),
    ],
    scratch_types=[
        pltpu.VMEM((_B_PER_W,), jnp.int32),
        pltpu.VMEM((_B_PER_W,), jnp.int32),
        pltpu.VMEM((_PANEL_H, PANEL), jnp.float32),
        pltpu.VMEM((_PANEL_H, PANEL), jnp.float32),
        pltpu.VMEM((_PANEL_H, PANEL), jnp.float32),
        pltpu.VMEM((_PANEL_H, PANEL), jnp.float32),
        pltpu.VMEM((_PANEL_H, PANEL), jnp.float32),
        pltpu.VMEM((_PANEL_H, PANEL), jnp.float32),
        pltpu.VMEM((_PANEL_H, PANEL), jnp.float32),
        pltpu.VMEM((_PANEL_H, PANEL), jnp.float32),
        pltpu.VMEM((_PANEL_H, _STAGE_W), jnp.float32),
        pltpu.SemaphoreType.DMA,
        pltpu.SemaphoreType.DMA,
        pltpu.SemaphoreType.DMA,
        pltpu.SemaphoreType.DMA,
        pltpu.SemaphoreType.DMA,
        pltpu.SemaphoreType.DMA,
        pltpu.SemaphoreType.DMA,
        pltpu.SemaphoreType.DMA,
    ],
    compiler_params=pltpu.CompilerParams(needs_layout_passes=False),
)(_gather_body)


def _mlp_body(euT_ref, eiT_ref, w1a_ref, w1b_ref, b1_ref, w2_ref, b2_ref,
              w3_ref, b3_ref, wp_ref, bp_ref, out_ref):
    h = w1a_ref[...] @ euT_ref[...] + w1b_ref[...] @ eiT_ref[...] + b1_ref[...]
    h = jnp.maximum(h, 0.0)
    h = jnp.maximum(w2_ref[...] @ h + b2_ref[...], 0.0)
    h = jnp.maximum(w3_ref[...] @ h + b3_ref[...], 0.0)
    logit = wp_ref[...] @ h + bp_ref[...]
    out_ref[...] = jax.nn.sigmoid(logit)


def _mlp(euT, eiT, w1a, w1b, b1, w2, b2, w3, b3, wp, bp, blk=4096):
    n_blocks = BATCH // blk

    def full(shape):
        zeros = (0,) * len(shape)
        return pl.BlockSpec(shape, lambda i: zeros)

    return pl.pallas_call(
        _mlp_body,
        grid=(n_blocks,),
        in_specs=[
            pl.BlockSpec((EMBED_DIM, blk), lambda i: (0, i)),
            pl.BlockSpec((EMBED_DIM, blk), lambda i: (0, i)),
            full(w1a.shape),
            full(w1b.shape),
            full(b1.shape),
            full(w2.shape),
            full(b2.shape),
            full(w3.shape),
            full(b3.shape),
            full(wp.shape),
            full(bp.shape),
        ],
        out_specs=pl.BlockSpec((1, blk), lambda i: (0, i)),
        out_shape=jax.ShapeDtypeStruct((1, BATCH), jnp.float32),
    )(euT, eiT, w1a, w1b, b1, w2, b2, w3, b3, wp, bp)


def kernel(user, item, embed_user, embed_item, W1, b1, W2, b2, W3, b3, Wp, bp):
    u = user.astype(jnp.int32)
    it = item.astype(jnp.int32)
    euT, eiT = _sc_gather(embed_user.T, embed_item.T, u, it)
    out = _mlp(
        euT, eiT,
        W1[:, :EMBED_DIM], W1[:, EMBED_DIM:], b1.reshape(-1, 1),
        W2, b2.reshape(-1, 1), W3, b3.reshape(-1, 1), Wp, bp.reshape(1, 1),
    )
    return out.reshape(-1)
